# Initial kernel scaffold; baseline (speedup 1.0000x reference)
import jax
import jax.numpy as jnp
from jax.experimental import pallas as pl


def kernel(x, edge_index, params):
    raise NotImplementedError("write your pallas kernel here")


# speedup vs baseline: 2.2046x; 2.2046x over previous
import functools

import jax
import jax.numpy as jnp
from jax import lax
from jax.experimental import pallas as pl
from jax.experimental.pallas import tpu as pltpu
from jax.experimental.pallas import tpu_sc as plsc

N_NODES = 50000
N_HEDGES = 50000
N_INC = 800000
EPSV = 1e-5

NC = 2
NS = 16
N_PAD = 51200
INC_PER_TILE = N_INC // NS
ROWS_PER_TILE = N_PAD // NS
EXP_ROWS = 80

DEG_W = 8
DEG_K = 200
DEG_ITERS = INC_PER_TILE // DEG_K


def _make_stage(n_chunks_per_sc, fc, chunk_k, exp_rows=EXP_ROWS):
  nct = n_chunks_per_sc * NC
  n_iters = INC_PER_TILE // chunk_k
  n_exp = ROWS_PER_TILE // exp_rows
  mesh = plsc.VectorSubcoreMesh(core_axis_name="c", subcore_axis_name="s")

  @functools.partial(
      pl.kernel,
      out_type=jax.ShapeDtypeStruct((nct, N_PAD, fc), jnp.float32),
      mesh=mesh,
      scratch_types=[
          pltpu.VMEM_SHARED((N_PAD, fc), jnp.float32),
          pltpu.SemaphoreType.DMA,
      ],
      compiler_params=pltpu.CompilerParams(use_tc_tiling_on_sc=False),
  )
  def stage(table_hbm, gidx_hbm, sidx_hbm, zeros_hbm, out_hbm, acc, sem):
    c = lax.axis_index("c")
    s = lax.axis_index("s")
    row0 = s * ROWS_PER_TILE
    def scoped(idx_v, data_v):
     gi_v = idx_v.at[pl.ds(0, chunk_k)]
     si_v = idx_v.at[pl.ds(chunk_k, chunk_k)]
     rows_v = data_v.at[pl.ds(0, chunk_k)]
     exp_v = data_v.at[pl.ds(chunk_k, exp_rows)]
     for q in range(n_chunks_per_sc):
      chunk = c * n_chunks_per_sc + q
      pltpu.sync_copy(zeros_hbm, exp_v)
      for j in range(n_exp):
        pltpu.sync_copy(exp_v, acc.at[pl.ds(row0 + j * exp_rows, exp_rows)])
      plsc.subcore_barrier()
      def body(g, _):
        base = s * INC_PER_TILE + g * chunk_k
        pltpu.sync_copy(gidx_hbm.at[pl.ds(base, chunk_k)], gi_v)
        pltpu.sync_copy(sidx_hbm.at[pl.ds(base, chunk_k)], si_v)
        pltpu.async_copy(table_hbm.at[chunk].at[gi_v], rows_v, sem).wait()
        pltpu.sync_copy(rows_v, acc.at[si_v], add=True)
        return 0
      lax.fori_loop(0, n_iters, body, 0)
      plsc.subcore_barrier()
      for j in range(n_exp):
        r = row0 + j * exp_rows
        pltpu.sync_copy(acc.at[pl.ds(r, exp_rows)], exp_v)
        pltpu.sync_copy(exp_v, out_hbm.at[chunk].at[pl.ds(r, exp_rows)])
    pl.run_scoped(
        scoped,
        pltpu.VMEM((2 * chunk_k,), jnp.int32),
        pltpu.VMEM((chunk_k + exp_rows, fc), jnp.float32),
    )

  return stage


def _make_degrees():
  mesh = plsc.VectorSubcoreMesh(core_axis_name="c", subcore_axis_name="s")

  @functools.partial(
      pl.kernel,
      out_type=jax.ShapeDtypeStruct((NC, N_PAD, DEG_W), jnp.float32),
      mesh=mesh,
      scratch_types=[
          pltpu.VMEM_SHARED((N_PAD, DEG_W), jnp.float32),
      ],
      compiler_params=pltpu.CompilerParams(use_tc_tiling_on_sc=False),
  )
  def degrees(src_hbm, hed_hbm, ones_hbm, zeros_hbm, out_hbm, acc):
    c = lax.axis_index("c")
    s = lax.axis_index("s")
    row0 = s * ROWS_PER_TILE
    def scoped(si_v, f_v):
      ones_v = f_v.at[pl.ds(0, DEG_K)]
      exp_v = f_v.at[pl.ds(DEG_K, EXP_ROWS)]
      pltpu.sync_copy(ones_hbm, ones_v)
      pltpu.sync_copy(zeros_hbm, exp_v)
      for j in range(ROWS_PER_TILE // EXP_ROWS):
        pltpu.sync_copy(exp_v, acc.at[pl.ds(row0 + j * EXP_ROWS, EXP_ROWS)])
      plsc.subcore_barrier()
      def body(g, _):
        base = s * INC_PER_TILE + g * DEG_K
        @pl.when(c == 0)
        def _():
          pltpu.sync_copy(src_hbm.at[pl.ds(base, DEG_K)], si_v)
        @pl.when(c == 1)
        def _():
          pltpu.sync_copy(hed_hbm.at[pl.ds(base, DEG_K)], si_v)
        pltpu.sync_copy(ones_v, acc.at[si_v], add=True)
        return 0
      lax.fori_loop(0, DEG_ITERS, body, 0)
      plsc.subcore_barrier()
      for j in range(ROWS_PER_TILE // EXP_ROWS):
        r = row0 + j * EXP_ROWS
        pltpu.sync_copy(acc.at[pl.ds(r, EXP_ROWS)], exp_v)
        pltpu.sync_copy(exp_v, out_hbm.at[c].at[pl.ds(r, EXP_ROWS)])
    pl.run_scoped(
        scoped,
        pltpu.VMEM((DEG_K,), jnp.int32),
        pltpu.VMEM((DEG_K + EXP_ROWS, DEG_W), jnp.float32),
    )

  return degrees


_TC_BLK = 1600


def _split_x(x):
  n = x.shape[0]
  blk = 2000

  def body(x_ref, o_ref):
    for k in range(4):
      o_ref[k] = x_ref[:, 24 * k:24 * (k + 1)]

  return pl.pallas_call(
      body,
      grid=(n // blk,),
      in_specs=[pl.BlockSpec((blk, 96), lambda j: (j, 0))],
      out_specs=pl.BlockSpec((4, blk, 24), lambda j: (0, j, 0)),
      out_shape=jax.ShapeDtypeStruct((4, n, 24), jnp.float32),
  )(x)


def _scale_tc(e_raw, b_raw):
  nct, n, fc = e_raw.shape

  def body(e_ref, b_ref, o_ref):
    b = b_ref[:, 0:1]
    binv = jnp.where(b > 0, 1.0 / b, 0.0)
    o_ref[...] = e_ref[...] * binv[None]

  return pl.pallas_call(
      body,
      grid=(n // _TC_BLK,),
      in_specs=[
          pl.BlockSpec((nct, _TC_BLK, fc), lambda j: (0, j, 0)),
          pl.BlockSpec((_TC_BLK, DEG_W), lambda j: (j, 0)),
      ],
      out_specs=pl.BlockSpec((nct, _TC_BLK, fc), lambda j: (0, j, 0)),
      out_shape=jax.ShapeDtypeStruct((nct, n, fc), jnp.float32),
  )(e_raw, b_raw)


def _layer_tc(y_raw, d_raw, w, bias, dout):
  nct, n, fc = y_raw.shape
  half = dout // 2

  def body(y_ref, d_ref, w_ref, b_ref, o_ref):
    d = d_ref[:, 0:1]
    dinv = jnp.where(d > 0, 1.0 / d, 0.0)
    acc = jnp.broadcast_to(b_ref[0], (_TC_BLK, 2 * dout)).astype(jnp.float32)
    for k in range(nct):
      acc = acc + jnp.dot(y_ref[k] * dinv, w_ref[k],
                          preferred_element_type=jnp.float32)
    h = jax.nn.relu(acc[:, :dout]) + acc[:, dout:]
    o_ref[0] = h[:, :half]
    o_ref[1] = h[:, half:]

  return pl.pallas_call(
      body,
      grid=(n // _TC_BLK,),
      in_specs=[
          pl.BlockSpec((nct, _TC_BLK, fc), lambda j: (0, j, 0)),
          pl.BlockSpec((_TC_BLK, DEG_W), lambda j: (j, 0)),
          pl.BlockSpec((nct, fc, 2 * dout), lambda j: (0, 0, 0)),
          pl.BlockSpec((1, 2 * dout), lambda j: (0, 0)),
      ],
      out_specs=pl.BlockSpec((2, _TC_BLK, half), lambda j: (0, j, 0)),
      out_shape=jax.ShapeDtypeStruct((2, n, half), jnp.float32),
  )(y_raw, d_raw, w, bias)


def _final_tc(y8_raw, y1_raw, d_raw, w8, w1, bias):
  nct8, n, fc8 = y8_raw.shape
  nct1, _, fc1 = y1_raw.shape
  dout = w8.shape[-1]

  def body(y8_ref, y1_ref, d_ref, w8_ref, w1_ref, b_ref, o_ref):
    d = d_ref[:, 0:1]
    dinv = jnp.where(d > 0, 1.0 / d, 0.0)
    acc = jnp.broadcast_to(b_ref[0], (2000, dout)).astype(jnp.float32)
    for k in range(nct8):
      acc = acc + jnp.dot(y8_ref[k] * dinv, w8_ref[k],
                          preferred_element_type=jnp.float32)
    for k in range(nct1):
      acc = acc + jnp.dot(y1_ref[k] * dinv, w1_ref[k],
                          preferred_element_type=jnp.float32)
    o_ref[...] = acc

  blk = 2000
  return pl.pallas_call(
      body,
      grid=(N_NODES // blk,),
      in_specs=[
          pl.BlockSpec((nct8, blk, fc8), lambda j: (0, j, 0)),
          pl.BlockSpec((nct1, blk, fc1), lambda j: (0, j, 0)),
          pl.BlockSpec((blk, DEG_W), lambda j: (j, 0)),
          pl.BlockSpec((nct8, fc8, dout), lambda j: (0, 0, 0)),
          pl.BlockSpec((nct1, fc1, dout), lambda j: (0, 0, 0)),
          pl.BlockSpec((1, dout), lambda j: (0, 0)),
      ],
      out_specs=pl.BlockSpec((blk, dout), lambda j: (j, 0)),
      out_shape=jax.ShapeDtypeStruct((N_NODES, dout), jnp.float32),
  )(y8_raw, y1_raw, d_raw, w8, w1, bias)


_stage_96 = _make_stage(2, 24, 80, 40)
_stage_80 = _make_stage(1, 40, 40, 16)
_degrees = _make_degrees()


def _sparse_pass(stage_fn, v_chunks, src, hed, zeros, b_raw):
  e_raw = stage_fn(v_chunks, src, hed, zeros)
  e_s = _scale_tc(e_raw, b_raw)
  return stage_fn(e_s, hed, src, zeros)


def kernel(x, edge_index, params):
  src = edge_index[0]
  hed = edge_index[1]

  cbn = 1.0 / jnp.sqrt(jnp.float32(1.0 + EPSV))
  ws, bs = [], []
  for i in range(1, 8):
    g = params['bn%d_g' % i] * cbn
    wc = params['conv%d_W' % i] * g[None, :]
    bc = params['conv%d_b' % i] * g + params['bn%d_b' % i]
    w = jnp.concatenate([wc, params['sk%d_W' % i]], axis=1)
    b = jnp.concatenate([bc, params['sk%d_b' % i]])
    fin = w.shape[0]
    nct = 4 if fin == 96 else 2
    ws.append(w.reshape(nct, fin // nct, w.shape[1]))
    bs.append(b.reshape(1, -1))

  wm = params['conv_mix_W']
  w8 = wm[:80].reshape(2, 40, 48)
  w1 = wm[80:].reshape(4, 24, 48)
  bm = params['conv_mix_b'].reshape(1, -1)

  x_chunks = _split_x(x)

  z24 = jnp.zeros((40, 24), jnp.float32)
  z40 = jnp.zeros((16, 40), jnp.float32)
  z8 = jnp.zeros((EXP_ROWS, DEG_W), jnp.float32)
  ones8 = jnp.ones((DEG_K, DEG_W), jnp.float32)

  deg = _degrees(src, hed, ones8, z8)
  d_raw = deg[0]
  b_raw = deg[1]

  y1_raw = _sparse_pass(_stage_96, x_chunks, src, hed, z24, b_raw)
  h = _layer_tc(y1_raw, d_raw, ws[0], bs[0], 80)

  for i in range(1, 7):
    y = _sparse_pass(_stage_80, h, src, hed, z40, b_raw)
    h = _layer_tc(y, d_raw, ws[i], bs[i], 80)
  y8_raw = _sparse_pass(_stage_80, h, src, hed, z40, b_raw)
  return _final_tc(y8_raw, y1_raw, d_raw, w8, w1, bm)


# speedup vs baseline: 4.4161x; 2.0031x over previous
import functools

import jax
import jax.numpy as jnp
from jax import lax
from jax.experimental import pallas as pl
from jax.experimental.pallas import tpu as pltpu
from jax.experimental.pallas import tpu_sc as plsc

N_NODES = 50000
N_HEDGES = 50000
N_INC = 800000
EPSV = 1e-5

NC = 2
NS = 16
N_PAD = 50176
INC_PER_TILE = N_INC // NS
ROWS_PER_TILE = N_PAD // NS
EXP_ROWS = 64

DEG_W = 8
DEG_K = 200
DEG_IB = 2000


def _make_stage(n_chunks_per_sc, fc, ib, sub, exp_rows=EXP_ROWS):
  nct = n_chunks_per_sc * NC
  n_outer = INC_PER_TILE // ib
  n_inner = ib // sub
  n_exp = ROWS_PER_TILE // exp_rows
  mesh = plsc.VectorSubcoreMesh(core_axis_name="c", subcore_axis_name="s")

  @functools.partial(
      pl.kernel,
      out_type=jax.ShapeDtypeStruct((nct, N_PAD, fc), jnp.float32),
      mesh=mesh,
      scratch_types=[
          pltpu.VMEM_SHARED((N_PAD, fc), jnp.float32),
          pltpu.SemaphoreType.DMA,
      ],
      compiler_params=pltpu.CompilerParams(use_tc_tiling_on_sc=False),
  )
  def stage(table_hbm, gidx_hbm, sidx_hbm, zeros_hbm, out_hbm, acc, sem):
    c = lax.axis_index("c")
    s = lax.axis_index("s")
    row0 = s * ROWS_PER_TILE
    def scoped(idx_v, data_v):
     bufs = (data_v.at[pl.ds(0, sub)], data_v.at[pl.ds(sub, sub)])
     exp_v = data_v.at[pl.ds(0, exp_rows)]
     for q in range(n_chunks_per_sc):
      chunk = c * n_chunks_per_sc + q
      pltpu.sync_copy(zeros_hbm, exp_v)
      for j in range(n_exp):
        pltpu.sync_copy(exp_v, acc.at[pl.ds(row0 + j * exp_rows, exp_rows)])
      plsc.subcore_barrier()
      def body(t, _):
        base = s * INC_PER_TILE + t * ib
        pltpu.sync_copy(gidx_hbm.at[pl.ds(base, ib)], idx_v.at[pl.ds(0, ib)])
        pltpu.sync_copy(sidx_hbm.at[pl.ds(base, ib)], idx_v.at[pl.ds(ib, ib)])
        cp = pltpu.async_copy(
            table_hbm.at[chunk].at[idx_v.at[pl.ds(0, sub)]], bufs[0], sem)
        for j in range(n_inner):
          cp.wait()
          if j + 1 < n_inner:
            nxt = pltpu.async_copy(
                table_hbm.at[chunk].at[idx_v.at[pl.ds((j + 1) * sub, sub)]],
                bufs[(j + 1) % 2], sem)
          pltpu.sync_copy(bufs[j % 2],
                          acc.at[idx_v.at[pl.ds(ib + j * sub, sub)]], add=True)
          if j + 1 < n_inner:
            cp = nxt
        return 0
      lax.fori_loop(0, n_outer, body, 0)
      plsc.subcore_barrier()
      for j in range(n_exp):
        r = row0 + j * exp_rows
        pltpu.sync_copy(acc.at[pl.ds(r, exp_rows)], exp_v)
        pltpu.sync_copy(exp_v, out_hbm.at[chunk].at[pl.ds(r, exp_rows)])
    pl.run_scoped(
        scoped,
        pltpu.VMEM((2 * ib,), jnp.int32),
        pltpu.VMEM((max(2 * sub, exp_rows), fc), jnp.float32),
    )

  return stage


def _make_degrees():
  mesh = plsc.VectorSubcoreMesh(core_axis_name="c", subcore_axis_name="s")

  @functools.partial(
      pl.kernel,
      out_type=jax.ShapeDtypeStruct((NC, N_PAD, DEG_W), jnp.float32),
      mesh=mesh,
      scratch_types=[
          pltpu.VMEM_SHARED((N_PAD, DEG_W), jnp.float32),
      ],
      compiler_params=pltpu.CompilerParams(use_tc_tiling_on_sc=False),
  )
  def degrees(src_hbm, hed_hbm, ones_hbm, zeros_hbm, out_hbm, acc):
    c = lax.axis_index("c")
    s = lax.axis_index("s")
    row0 = s * ROWS_PER_TILE
    def scoped(si_v, f_v):
      ones_v = f_v.at[pl.ds(0, DEG_K)]
      exp_v = f_v.at[pl.ds(DEG_K, EXP_ROWS)]
      pltpu.sync_copy(ones_hbm, ones_v)
      pltpu.sync_copy(zeros_hbm, exp_v)
      for j in range(ROWS_PER_TILE // EXP_ROWS):
        pltpu.sync_copy(exp_v, acc.at[pl.ds(row0 + j * EXP_ROWS, EXP_ROWS)])
      plsc.subcore_barrier()
      def body(t, _):
        base = s * INC_PER_TILE + t * DEG_IB
        @pl.when(c == 0)
        def _():
          pltpu.sync_copy(src_hbm.at[pl.ds(base, DEG_IB)], si_v)
        @pl.when(c == 1)
        def _():
          pltpu.sync_copy(hed_hbm.at[pl.ds(base, DEG_IB)], si_v)
        for j in range(DEG_IB // DEG_K):
          pltpu.sync_copy(ones_v,
                          acc.at[si_v.at[pl.ds(j * DEG_K, DEG_K)]], add=True)
        return 0
      lax.fori_loop(0, INC_PER_TILE // DEG_IB, body, 0)
      plsc.subcore_barrier()
      for j in range(ROWS_PER_TILE // EXP_ROWS):
        r = row0 + j * EXP_ROWS
        pltpu.sync_copy(acc.at[pl.ds(r, EXP_ROWS)], exp_v)
        pltpu.sync_copy(exp_v, out_hbm.at[c].at[pl.ds(r, EXP_ROWS)])
    pl.run_scoped(
        scoped,
        pltpu.VMEM((DEG_IB,), jnp.int32),
        pltpu.VMEM((DEG_K + EXP_ROWS, DEG_W), jnp.float32),
    )

  return degrees


_TC_BLK = 896


def _split_x(x):
  n = x.shape[0]
  blk = 2000

  def body(x_ref, o_ref):
    for k in range(4):
      o_ref[k] = x_ref[:, 24 * k:24 * (k + 1)]

  return pl.pallas_call(
      body,
      grid=(n // blk,),
      in_specs=[pl.BlockSpec((blk, 96), lambda j: (j, 0))],
      out_specs=pl.BlockSpec((4, blk, 24), lambda j: (0, j, 0)),
      out_shape=jax.ShapeDtypeStruct((4, n, 24), jnp.float32),
  )(x)


def _scale_tc(e_raw, b_raw):
  nct, n, fc = e_raw.shape

  def body(e_ref, b_ref, o_ref):
    b = b_ref[:, 0:1]
    binv = jnp.where(b > 0, 1.0 / b, 0.0)
    o_ref[...] = e_ref[...] * binv[None]

  return pl.pallas_call(
      body,
      grid=(n // _TC_BLK,),
      in_specs=[
          pl.BlockSpec((nct, _TC_BLK, fc), lambda j: (0, j, 0)),
          pl.BlockSpec((_TC_BLK, DEG_W), lambda j: (j, 0)),
      ],
      out_specs=pl.BlockSpec((nct, _TC_BLK, fc), lambda j: (0, j, 0)),
      out_shape=jax.ShapeDtypeStruct((nct, n, fc), jnp.float32),
  )(e_raw, b_raw)


def _layer_tc(y_raw, d_raw, w, bias, dout):
  nct, n, fc = y_raw.shape
  half = dout // 2

  def body(y_ref, d_ref, w_ref, b_ref, o_ref):
    d = d_ref[:, 0:1]
    dinv = jnp.where(d > 0, 1.0 / d, 0.0)
    acc = jnp.broadcast_to(b_ref[0], (_TC_BLK, 2 * dout)).astype(jnp.float32)
    for k in range(nct):
      acc = acc + jnp.dot(y_ref[k] * dinv, w_ref[k],
                          preferred_element_type=jnp.float32)
    h = jax.nn.relu(acc[:, :dout]) + acc[:, dout:]
    o_ref[0] = h[:, :half]
    o_ref[1] = h[:, half:]

  return pl.pallas_call(
      body,
      grid=(n // _TC_BLK,),
      in_specs=[
          pl.BlockSpec((nct, _TC_BLK, fc), lambda j: (0, j, 0)),
          pl.BlockSpec((_TC_BLK, DEG_W), lambda j: (j, 0)),
          pl.BlockSpec((nct, fc, 2 * dout), lambda j: (0, 0, 0)),
          pl.BlockSpec((1, 2 * dout), lambda j: (0, 0)),
      ],
      out_specs=pl.BlockSpec((2, _TC_BLK, half), lambda j: (0, j, 0)),
      out_shape=jax.ShapeDtypeStruct((2, n, half), jnp.float32),
  )(y_raw, d_raw, w, bias)


def _final_tc(y8_raw, y1_raw, d_raw, w8, w1, bias):
  nct8, n, fc8 = y8_raw.shape
  nct1, _, fc1 = y1_raw.shape
  dout = w8.shape[-1]

  def body(y8_ref, y1_ref, d_ref, w8_ref, w1_ref, b_ref, o_ref):
    d = d_ref[:, 0:1]
    dinv = jnp.where(d > 0, 1.0 / d, 0.0)
    acc = jnp.broadcast_to(b_ref[0], (2000, dout)).astype(jnp.float32)
    for k in range(nct8):
      acc = acc + jnp.dot(y8_ref[k] * dinv, w8_ref[k],
                          preferred_element_type=jnp.float32)
    for k in range(nct1):
      acc = acc + jnp.dot(y1_ref[k] * dinv, w1_ref[k],
                          preferred_element_type=jnp.float32)
    o_ref[...] = acc

  blk = 2000
  return pl.pallas_call(
      body,
      grid=(N_NODES // blk,),
      in_specs=[
          pl.BlockSpec((nct8, blk, fc8), lambda j: (0, j, 0)),
          pl.BlockSpec((nct1, blk, fc1), lambda j: (0, j, 0)),
          pl.BlockSpec((blk, DEG_W), lambda j: (j, 0)),
          pl.BlockSpec((nct8, fc8, dout), lambda j: (0, 0, 0)),
          pl.BlockSpec((nct1, fc1, dout), lambda j: (0, 0, 0)),
          pl.BlockSpec((1, dout), lambda j: (0, 0)),
      ],
      out_specs=pl.BlockSpec((blk, dout), lambda j: (j, 0)),
      out_shape=jax.ShapeDtypeStruct((N_NODES, dout), jnp.float32),
  )(y8_raw, y1_raw, d_raw, w8, w1, bias)


_stage_96 = _make_stage(2, 24, 2000, 200)
_stage_80 = _make_stage(1, 40, 400, 40)
_degrees = _make_degrees()


def _sparse_pass(stage_fn, v_chunks, src, hed, zeros, b_raw):
  e_raw = stage_fn(v_chunks, src, hed, zeros)
  e_s = _scale_tc(e_raw, b_raw)
  return stage_fn(e_s, hed, src, zeros)


def kernel(x, edge_index, params):
  src = edge_index[0]
  hed = edge_index[1]

  cbn = 1.0 / jnp.sqrt(jnp.float32(1.0 + EPSV))
  ws, bs = [], []
  for i in range(1, 8):
    g = params['bn%d_g' % i] * cbn
    wc = params['conv%d_W' % i] * g[None, :]
    bc = params['conv%d_b' % i] * g + params['bn%d_b' % i]
    w = jnp.concatenate([wc, params['sk%d_W' % i]], axis=1)
    b = jnp.concatenate([bc, params['sk%d_b' % i]])
    fin = w.shape[0]
    nct = 4 if fin == 96 else 2
    ws.append(w.reshape(nct, fin // nct, w.shape[1]))
    bs.append(b.reshape(1, -1))

  wm = params['conv_mix_W']
  w8 = wm[:80].reshape(2, 40, 48)
  w1 = wm[80:].reshape(4, 24, 48)
  bm = params['conv_mix_b'].reshape(1, -1)

  x_chunks = _split_x(x)

  z24 = jnp.zeros((EXP_ROWS, 24), jnp.float32)
  z40 = jnp.zeros((EXP_ROWS, 40), jnp.float32)
  z8 = jnp.zeros((EXP_ROWS, DEG_W), jnp.float32)
  ones8 = jnp.ones((DEG_K, DEG_W), jnp.float32)

  deg = _degrees(src, hed, ones8, z8)
  d_raw = deg[0]
  b_raw = deg[1]

  y1_raw = _sparse_pass(_stage_96, x_chunks, src, hed, z24, b_raw)
  h = _layer_tc(y1_raw, d_raw, ws[0], bs[0], 80)

  for i in range(1, 7):
    y = _sparse_pass(_stage_80, h, src, hed, z40, b_raw)
    h = _layer_tc(y, d_raw, ws[i], bs[i], 80)
  y8_raw = _sparse_pass(_stage_80, h, src, hed, z40, b_raw)
  return _final_tc(y8_raw, y1_raw, d_raw, w8, w1, bm)


# speedup vs baseline: 4.7046x; 1.0653x over previous
import functools

import jax
import jax.numpy as jnp
from jax import lax
from jax.experimental import pallas as pl
from jax.experimental.pallas import tpu as pltpu
from jax.experimental.pallas import tpu_sc as plsc

N_NODES = 50000
N_HEDGES = 50000
N_INC = 800000
EPSV = 1e-5

NC = 2
NS = 16
N_PAD = 50176
INC_PER_TILE = N_INC // NS
ROWS_PER_TILE = N_PAD // NS
EXP_ROWS = 64

DEG_W = 8
DEG_K = 200
DEG_IB = 2000


def _make_stage(n_chunks_per_sc, fc, ib, sub, exp_rows=EXP_ROWS):
  nct = n_chunks_per_sc * NC
  n_outer = INC_PER_TILE // ib
  n_inner = ib // sub
  n_exp = ROWS_PER_TILE // exp_rows
  mesh = plsc.VectorSubcoreMesh(core_axis_name="c", subcore_axis_name="s")

  @functools.partial(
      pl.kernel,
      out_type=jax.ShapeDtypeStruct((nct, N_PAD, fc), jnp.float32),
      mesh=mesh,
      scratch_types=[
          pltpu.VMEM_SHARED((N_PAD, fc), jnp.float32),
          pltpu.SemaphoreType.DMA,
      ],
      compiler_params=pltpu.CompilerParams(use_tc_tiling_on_sc=False),
  )
  def stage(table_hbm, gidx_hbm, sidx_hbm, zeros_hbm, out_hbm, acc, sem):
    c = lax.axis_index("c")
    s = lax.axis_index("s")
    row0 = s * ROWS_PER_TILE
    def scoped(idx_v, data_v):
     bufs = (data_v.at[pl.ds(0, sub)], data_v.at[pl.ds(sub, sub)])
     exp_v = data_v.at[pl.ds(0, exp_rows)]
     for q in range(n_chunks_per_sc):
      chunk = c * n_chunks_per_sc + q
      pltpu.sync_copy(zeros_hbm, exp_v)
      for j in range(n_exp):
        pltpu.sync_copy(exp_v, acc.at[pl.ds(row0 + j * exp_rows, exp_rows)])
      plsc.subcore_barrier()
      def body(t, _):
        base = s * INC_PER_TILE + t * ib
        pltpu.sync_copy(gidx_hbm.at[pl.ds(base, ib)], idx_v.at[pl.ds(0, ib)])
        pltpu.sync_copy(sidx_hbm.at[pl.ds(base, ib)], idx_v.at[pl.ds(ib, ib)])
        cp = pltpu.async_copy(
            table_hbm.at[chunk].at[idx_v.at[pl.ds(0, sub)]], bufs[0], sem)
        for j in range(n_inner):
          cp.wait()
          if j + 1 < n_inner:
            nxt = pltpu.async_copy(
                table_hbm.at[chunk].at[idx_v.at[pl.ds((j + 1) * sub, sub)]],
                bufs[(j + 1) % 2], sem)
          pltpu.sync_copy(bufs[j % 2],
                          acc.at[idx_v.at[pl.ds(ib + j * sub, sub)]], add=True)
          if j + 1 < n_inner:
            cp = nxt
        return 0
      lax.fori_loop(0, n_outer, body, 0)
      plsc.subcore_barrier()
      for j in range(n_exp):
        r = row0 + j * exp_rows
        pltpu.sync_copy(acc.at[pl.ds(r, exp_rows)], exp_v)
        pltpu.sync_copy(exp_v, out_hbm.at[chunk].at[pl.ds(r, exp_rows)])
    pl.run_scoped(
        scoped,
        pltpu.VMEM((2 * ib,), jnp.int32),
        pltpu.VMEM((max(2 * sub, exp_rows), fc), jnp.float32),
    )

  return stage


def _make_degrees():
  mesh = plsc.VectorSubcoreMesh(core_axis_name="c", subcore_axis_name="s")

  @functools.partial(
      pl.kernel,
      out_type=jax.ShapeDtypeStruct((NC, N_PAD, DEG_W), jnp.float32),
      mesh=mesh,
      scratch_types=[
          pltpu.VMEM_SHARED((N_PAD, DEG_W), jnp.float32),
      ],
      compiler_params=pltpu.CompilerParams(use_tc_tiling_on_sc=False),
  )
  def degrees(src_hbm, hed_hbm, ones_hbm, zeros_hbm, out_hbm, acc):
    c = lax.axis_index("c")
    s = lax.axis_index("s")
    row0 = s * ROWS_PER_TILE
    def scoped(si_v, f_v):
      ones_v = f_v.at[pl.ds(0, DEG_K)]
      exp_v = f_v.at[pl.ds(DEG_K, EXP_ROWS)]
      pltpu.sync_copy(ones_hbm, ones_v)
      pltpu.sync_copy(zeros_hbm, exp_v)
      for j in range(ROWS_PER_TILE // EXP_ROWS):
        pltpu.sync_copy(exp_v, acc.at[pl.ds(row0 + j * EXP_ROWS, EXP_ROWS)])
      plsc.subcore_barrier()
      def body(t, _):
        base = s * INC_PER_TILE + t * DEG_IB
        @pl.when(c == 0)
        def _():
          pltpu.sync_copy(src_hbm.at[pl.ds(base, DEG_IB)], si_v)
        @pl.when(c == 1)
        def _():
          pltpu.sync_copy(hed_hbm.at[pl.ds(base, DEG_IB)], si_v)
        for j in range(DEG_IB // DEG_K):
          pltpu.sync_copy(ones_v,
                          acc.at[si_v.at[pl.ds(j * DEG_K, DEG_K)]], add=True)
        return 0
      lax.fori_loop(0, INC_PER_TILE // DEG_IB, body, 0)
      plsc.subcore_barrier()
      for j in range(ROWS_PER_TILE // EXP_ROWS):
        r = row0 + j * EXP_ROWS
        pltpu.sync_copy(acc.at[pl.ds(r, EXP_ROWS)], exp_v)
        pltpu.sync_copy(exp_v, out_hbm.at[c].at[pl.ds(r, EXP_ROWS)])
    pl.run_scoped(
        scoped,
        pltpu.VMEM((DEG_IB,), jnp.int32),
        pltpu.VMEM((DEG_K + EXP_ROWS, DEG_W), jnp.float32),
    )

  return degrees


_TC_BLK = 896


def _split_x(x):
  n = x.shape[0]
  blk = 2000

  def body(x_ref, o_ref):
    for k in range(4):
      o_ref[k] = x_ref[:, 24 * k:24 * (k + 1)]

  return pl.pallas_call(
      body,
      grid=(n // blk,),
      in_specs=[pl.BlockSpec((blk, 96), lambda j: (j, 0))],
      out_specs=pl.BlockSpec((4, blk, 24), lambda j: (0, j, 0)),
      out_shape=jax.ShapeDtypeStruct((4, n, 24), jnp.float32),
  )(x)


def _scale_tc(e_raw, b_raw):
  nct, n, fc = e_raw.shape

  def body(e_ref, b_ref, o_ref):
    b = b_ref[:, 0:1]
    binv = jnp.where(b > 0, 1.0 / b, 0.0)
    o_ref[...] = e_ref[...] * binv[None]

  return pl.pallas_call(
      body,
      grid=(n // _TC_BLK,),
      in_specs=[
          pl.BlockSpec((nct, _TC_BLK, fc), lambda j: (0, j, 0)),
          pl.BlockSpec((_TC_BLK, DEG_W), lambda j: (j, 0)),
      ],
      out_specs=pl.BlockSpec((nct, _TC_BLK, fc), lambda j: (0, j, 0)),
      out_shape=jax.ShapeDtypeStruct((nct, n, fc), jnp.float32),
  )(e_raw, b_raw)


def _layer_tc(y_raw, d_raw, w, bias, dout):
  nct, n, fc = y_raw.shape
  half = dout // 2

  def body(y_ref, d_ref, w_ref, b_ref, o_ref):
    d = d_ref[:, 0:1]
    dinv = jnp.where(d > 0, 1.0 / d, 0.0)
    acc = jnp.broadcast_to(b_ref[0], (_TC_BLK, 2 * dout)).astype(jnp.float32)
    for k in range(nct):
      acc = acc + jnp.dot(y_ref[k] * dinv, w_ref[k],
                          preferred_element_type=jnp.float32)
    h = jax.nn.relu(acc[:, :dout]) + acc[:, dout:]
    o_ref[0] = h[:, :half]
    o_ref[1] = h[:, half:]

  return pl.pallas_call(
      body,
      grid=(n // _TC_BLK,),
      in_specs=[
          pl.BlockSpec((nct, _TC_BLK, fc), lambda j: (0, j, 0)),
          pl.BlockSpec((_TC_BLK, DEG_W), lambda j: (j, 0)),
          pl.BlockSpec((nct, fc, 2 * dout), lambda j: (0, 0, 0)),
          pl.BlockSpec((1, 2 * dout), lambda j: (0, 0)),
      ],
      out_specs=pl.BlockSpec((2, _TC_BLK, half), lambda j: (0, j, 0)),
      out_shape=jax.ShapeDtypeStruct((2, n, half), jnp.float32),
  )(y_raw, d_raw, w, bias)


def _final_tc(y8_raw, y1_raw, d_raw, w8, w1, bias):
  nct8, n, fc8 = y8_raw.shape
  nct1, _, fc1 = y1_raw.shape
  dout = w8.shape[-1]

  def body(y8_ref, y1_ref, d_ref, w8_ref, w1_ref, b_ref, o_ref):
    d = d_ref[:, 0:1]
    dinv = jnp.where(d > 0, 1.0 / d, 0.0)
    acc = jnp.broadcast_to(b_ref[0], (2000, dout)).astype(jnp.float32)
    for k in range(nct8):
      acc = acc + jnp.dot(y8_ref[k] * dinv, w8_ref[k],
                          preferred_element_type=jnp.float32)
    for k in range(nct1):
      acc = acc + jnp.dot(y1_ref[k] * dinv, w1_ref[k],
                          preferred_element_type=jnp.float32)
    o_ref[...] = acc

  blk = 2000
  return pl.pallas_call(
      body,
      grid=(N_NODES // blk,),
      in_specs=[
          pl.BlockSpec((nct8, blk, fc8), lambda j: (0, j, 0)),
          pl.BlockSpec((nct1, blk, fc1), lambda j: (0, j, 0)),
          pl.BlockSpec((blk, DEG_W), lambda j: (j, 0)),
          pl.BlockSpec((nct8, fc8, dout), lambda j: (0, 0, 0)),
          pl.BlockSpec((nct1, fc1, dout), lambda j: (0, 0, 0)),
          pl.BlockSpec((1, dout), lambda j: (0, 0)),
      ],
      out_specs=pl.BlockSpec((blk, dout), lambda j: (j, 0)),
      out_shape=jax.ShapeDtypeStruct((N_NODES, dout), jnp.float32),
  )(y8_raw, y1_raw, d_raw, w8, w1, bias)


_stage_96 = _make_stage(2, 24, 2000, 200)
_stage_80 = _make_stage(1, 40, 1000, 40)
_degrees = _make_degrees()


def _sparse_pass(stage_fn, v_chunks, src, hed, zeros, b_raw):
  e_raw = stage_fn(v_chunks, src, hed, zeros)
  e_s = _scale_tc(e_raw, b_raw)
  return stage_fn(e_s, hed, src, zeros)


def kernel(x, edge_index, params):
  src = edge_index[0]
  hed = edge_index[1]

  cbn = 1.0 / jnp.sqrt(jnp.float32(1.0 + EPSV))
  ws, bs = [], []
  for i in range(1, 8):
    g = params['bn%d_g' % i] * cbn
    wc = params['conv%d_W' % i] * g[None, :]
    bc = params['conv%d_b' % i] * g + params['bn%d_b' % i]
    w = jnp.concatenate([wc, params['sk%d_W' % i]], axis=1)
    b = jnp.concatenate([bc, params['sk%d_b' % i]])
    fin = w.shape[0]
    nct = 4 if fin == 96 else 2
    ws.append(w.reshape(nct, fin // nct, w.shape[1]))
    bs.append(b.reshape(1, -1))

  wm = params['conv_mix_W']
  w8 = wm[:80].reshape(2, 40, 48)
  w1 = wm[80:].reshape(4, 24, 48)
  bm = params['conv_mix_b'].reshape(1, -1)

  x_chunks = _split_x(x)

  z24 = jnp.zeros((EXP_ROWS, 24), jnp.float32)
  z40 = jnp.zeros((EXP_ROWS, 40), jnp.float32)
  z8 = jnp.zeros((EXP_ROWS, DEG_W), jnp.float32)
  ones8 = jnp.ones((DEG_K, DEG_W), jnp.float32)

  deg = _degrees(src, hed, ones8, z8)
  d_raw = deg[0]
  b_raw = deg[1]

  y1_raw = _sparse_pass(_stage_96, x_chunks, src, hed, z24, b_raw)
  h = _layer_tc(y1_raw, d_raw, ws[0], bs[0], 80)

  for i in range(1, 7):
    y = _sparse_pass(_stage_80, h, src, hed, z40, b_raw)
    h = _layer_tc(y, d_raw, ws[i], bs[i], 80)
  y8_raw = _sparse_pass(_stage_80, h, src, hed, z40, b_raw)
  return _final_tc(y8_raw, y1_raw, d_raw, w8, w1, bm)


# speedup vs baseline: 4.8567x; 1.0323x over previous
import functools

import jax
import jax.numpy as jnp
from jax import lax
from jax.experimental import pallas as pl
from jax.experimental.pallas import tpu as pltpu
from jax.experimental.pallas import tpu_sc as plsc

N_NODES = 50000
N_HEDGES = 50000
N_INC = 800000
EPSV = 1e-5

NC = 2
NS = 16
N_PAD = 50176
INC_PER_TILE = N_INC // NS
ROWS_PER_TILE = N_PAD // NS
EXP_ROWS = 64

DEG_W = 8
DEG_K = 200
DEG_IB = 2000


def _make_stage(n_chunks_per_sc, fc, ib, sub, exp_rows=EXP_ROWS):
  nct = n_chunks_per_sc * NC
  n_outer = INC_PER_TILE // ib
  n_inner = ib // sub
  n_exp = ROWS_PER_TILE // exp_rows
  assert n_outer % 2 == 0
  mesh = plsc.VectorSubcoreMesh(core_axis_name="c", subcore_axis_name="s")

  @functools.partial(
      pl.kernel,
      out_type=jax.ShapeDtypeStruct((nct, N_PAD, fc), jnp.float32),
      mesh=mesh,
      scratch_types=[
          pltpu.VMEM_SHARED((N_PAD, fc), jnp.float32),
          pltpu.SemaphoreType.DMA,
          pltpu.SemaphoreType.DMA,
      ],
      compiler_params=pltpu.CompilerParams(use_tc_tiling_on_sc=False),
  )
  def stage(table_hbm, gidx_hbm, sidx_hbm, zeros_hbm, out_hbm, acc, sem, semi):
    c = lax.axis_index("c")
    s = lax.axis_index("s")
    row0 = s * ROWS_PER_TILE
    def scoped(idx_v, data_v):
     bufs = (data_v.at[pl.ds(0, sub)], data_v.at[pl.ds(sub, sub)])
     exp_v = data_v.at[pl.ds(0, exp_rows)]
     blks = (idx_v.at[0], idx_v.at[1])
     for q in range(n_chunks_per_sc):
      chunk = c * n_chunks_per_sc + q
      pltpu.sync_copy(zeros_hbm, exp_v)
      for j in range(n_exp):
        pltpu.sync_copy(exp_v, acc.at[pl.ds(row0 + j * exp_rows, exp_rows)])
      plsc.subcore_barrier()
      base0 = s * INC_PER_TILE
      pltpu.sync_copy(gidx_hbm.at[pl.ds(base0, ib)], blks[0].at[pl.ds(0, ib)])
      pltpu.sync_copy(sidx_hbm.at[pl.ds(base0, ib)], blks[0].at[pl.ds(ib, ib)])
      def process(blk):
        cp = pltpu.async_copy(
            table_hbm.at[chunk].at[blk.at[pl.ds(0, sub)]], bufs[0], sem)
        for j in range(n_inner):
          cp.wait()
          if j + 1 < n_inner:
            nxt = pltpu.async_copy(
                table_hbm.at[chunk].at[blk.at[pl.ds((j + 1) * sub, sub)]],
                bufs[(j + 1) % 2], sem)
          pltpu.sync_copy(bufs[j % 2],
                          acc.at[blk.at[pl.ds(ib + j * sub, sub)]], add=True)
          if j + 1 < n_inner:
            cp = nxt
      def body(u, _):
        b1 = base0 + (2 * u + 1) * ib
        c1 = pltpu.async_copy(gidx_hbm.at[pl.ds(b1, ib)],
                              blks[1].at[pl.ds(0, ib)], semi)
        c2 = pltpu.async_copy(sidx_hbm.at[pl.ds(b1, ib)],
                              blks[1].at[pl.ds(ib, ib)], semi)
        process(blks[0])
        c1.wait()
        c2.wait()
        b2 = base0 + lax.rem(2 * u + 2, n_outer) * ib
        c3 = pltpu.async_copy(gidx_hbm.at[pl.ds(b2, ib)],
                              blks[0].at[pl.ds(0, ib)], semi)
        c4 = pltpu.async_copy(sidx_hbm.at[pl.ds(b2, ib)],
                              blks[0].at[pl.ds(ib, ib)], semi)
        process(blks[1])
        c3.wait()
        c4.wait()
        return 0
      lax.fori_loop(0, n_outer // 2, body, 0)
      plsc.subcore_barrier()
      for j in range(n_exp):
        r = row0 + j * exp_rows
        pltpu.sync_copy(acc.at[pl.ds(r, exp_rows)], exp_v)
        pltpu.sync_copy(exp_v, out_hbm.at[chunk].at[pl.ds(r, exp_rows)])
    pl.run_scoped(
        scoped,
        pltpu.VMEM((2, 2 * ib), jnp.int32),
        pltpu.VMEM((max(2 * sub, exp_rows), fc), jnp.float32),
    )

  return stage


def _make_degrees():
  mesh = plsc.VectorSubcoreMesh(core_axis_name="c", subcore_axis_name="s")

  @functools.partial(
      pl.kernel,
      out_type=jax.ShapeDtypeStruct((NC, N_PAD, DEG_W), jnp.float32),
      mesh=mesh,
      scratch_types=[
          pltpu.VMEM_SHARED((N_PAD, DEG_W), jnp.float32),
      ],
      compiler_params=pltpu.CompilerParams(use_tc_tiling_on_sc=False),
  )
  def degrees(src_hbm, hed_hbm, ones_hbm, zeros_hbm, out_hbm, acc):
    c = lax.axis_index("c")
    s = lax.axis_index("s")
    row0 = s * ROWS_PER_TILE
    def scoped(si_v, f_v):
      ones_v = f_v.at[pl.ds(0, DEG_K)]
      exp_v = f_v.at[pl.ds(DEG_K, EXP_ROWS)]
      pltpu.sync_copy(ones_hbm, ones_v)
      pltpu.sync_copy(zeros_hbm, exp_v)
      for j in range(ROWS_PER_TILE // EXP_ROWS):
        pltpu.sync_copy(exp_v, acc.at[pl.ds(row0 + j * EXP_ROWS, EXP_ROWS)])
      plsc.subcore_barrier()
      def body(t, _):
        base = s * INC_PER_TILE + t * DEG_IB
        @pl.when(c == 0)
        def _():
          pltpu.sync_copy(src_hbm.at[pl.ds(base, DEG_IB)], si_v)
        @pl.when(c == 1)
        def _():
          pltpu.sync_copy(hed_hbm.at[pl.ds(base, DEG_IB)], si_v)
        for j in range(DEG_IB // DEG_K):
          pltpu.sync_copy(ones_v,
                          acc.at[si_v.at[pl.ds(j * DEG_K, DEG_K)]], add=True)
        return 0
      lax.fori_loop(0, INC_PER_TILE // DEG_IB, body, 0)
      plsc.subcore_barrier()
      for j in range(ROWS_PER_TILE // EXP_ROWS):
        r = row0 + j * EXP_ROWS
        pltpu.sync_copy(acc.at[pl.ds(r, EXP_ROWS)], exp_v)
        pltpu.sync_copy(exp_v, out_hbm.at[c].at[pl.ds(r, EXP_ROWS)])
    pl.run_scoped(
        scoped,
        pltpu.VMEM((DEG_IB,), jnp.int32),
        pltpu.VMEM((DEG_K + EXP_ROWS, DEG_W), jnp.float32),
    )

  return degrees


_TC_BLK = 896


def _split_x(x):
  n = x.shape[0]
  blk = 2000

  def body(x_ref, o_ref):
    for k in range(4):
      o_ref[k] = x_ref[:, 24 * k:24 * (k + 1)]

  return pl.pallas_call(
      body,
      grid=(n // blk,),
      in_specs=[pl.BlockSpec((blk, 96), lambda j: (j, 0))],
      out_specs=pl.BlockSpec((4, blk, 24), lambda j: (0, j, 0)),
      out_shape=jax.ShapeDtypeStruct((4, n, 24), jnp.float32),
  )(x)


def _scale_tc(e_raw, b_raw):
  nct, n, fc = e_raw.shape

  def body(e_ref, b_ref, o_ref):
    b = b_ref[:, 0:1]
    binv = jnp.where(b > 0, 1.0 / b, 0.0)
    o_ref[...] = e_ref[...] * binv[None]

  return pl.pallas_call(
      body,
      grid=(n // _TC_BLK,),
      in_specs=[
          pl.BlockSpec((nct, _TC_BLK, fc), lambda j: (0, j, 0)),
          pl.BlockSpec((_TC_BLK, DEG_W), lambda j: (j, 0)),
      ],
      out_specs=pl.BlockSpec((nct, _TC_BLK, fc), lambda j: (0, j, 0)),
      out_shape=jax.ShapeDtypeStruct((nct, n, fc), jnp.float32),
  )(e_raw, b_raw)


def _layer_tc(y_raw, d_raw, w, bias, dout):
  nct, n, fc = y_raw.shape
  half = dout // 2

  def body(y_ref, d_ref, w_ref, b_ref, o_ref):
    d = d_ref[:, 0:1]
    dinv = jnp.where(d > 0, 1.0 / d, 0.0)
    acc = jnp.broadcast_to(b_ref[0], (_TC_BLK, 2 * dout)).astype(jnp.float32)
    for k in range(nct):
      acc = acc + jnp.dot(y_ref[k] * dinv, w_ref[k],
                          preferred_element_type=jnp.float32)
    h = jax.nn.relu(acc[:, :dout]) + acc[:, dout:]
    o_ref[0] = h[:, :half]
    o_ref[1] = h[:, half:]

  return pl.pallas_call(
      body,
      grid=(n // _TC_BLK,),
      in_specs=[
          pl.BlockSpec((nct, _TC_BLK, fc), lambda j: (0, j, 0)),
          pl.BlockSpec((_TC_BLK, DEG_W), lambda j: (j, 0)),
          pl.BlockSpec((nct, fc, 2 * dout), lambda j: (0, 0, 0)),
          pl.BlockSpec((1, 2 * dout), lambda j: (0, 0)),
      ],
      out_specs=pl.BlockSpec((2, _TC_BLK, half), lambda j: (0, j, 0)),
      out_shape=jax.ShapeDtypeStruct((2, n, half), jnp.float32),
  )(y_raw, d_raw, w, bias)


def _final_tc(y8_raw, y1_raw, d_raw, w8, w1, bias):
  nct8, n, fc8 = y8_raw.shape
  nct1, _, fc1 = y1_raw.shape
  dout = w8.shape[-1]

  def body(y8_ref, y1_ref, d_ref, w8_ref, w1_ref, b_ref, o_ref):
    d = d_ref[:, 0:1]
    dinv = jnp.where(d > 0, 1.0 / d, 0.0)
    acc = jnp.broadcast_to(b_ref[0], (2000, dout)).astype(jnp.float32)
    for k in range(nct8):
      acc = acc + jnp.dot(y8_ref[k] * dinv, w8_ref[k],
                          preferred_element_type=jnp.float32)
    for k in range(nct1):
      acc = acc + jnp.dot(y1_ref[k] * dinv, w1_ref[k],
                          preferred_element_type=jnp.float32)
    o_ref[...] = acc

  blk = 2000
  return pl.pallas_call(
      body,
      grid=(N_NODES // blk,),
      in_specs=[
          pl.BlockSpec((nct8, blk, fc8), lambda j: (0, j, 0)),
          pl.BlockSpec((nct1, blk, fc1), lambda j: (0, j, 0)),
          pl.BlockSpec((blk, DEG_W), lambda j: (j, 0)),
          pl.BlockSpec((nct8, fc8, dout), lambda j: (0, 0, 0)),
          pl.BlockSpec((nct1, fc1, dout), lambda j: (0, 0, 0)),
          pl.BlockSpec((1, dout), lambda j: (0, 0)),
      ],
      out_specs=pl.BlockSpec((blk, dout), lambda j: (j, 0)),
      out_shape=jax.ShapeDtypeStruct((N_NODES, dout), jnp.float32),
  )(y8_raw, y1_raw, d_raw, w8, w1, bias)


_stage_96 = _make_stage(2, 24, 1000, 200)
_stage_80 = _make_stage(1, 40, 200, 40)
_degrees = _make_degrees()


def _sparse_pass(stage_fn, v_chunks, src, hed, zeros, b_raw):
  e_raw = stage_fn(v_chunks, src, hed, zeros)
  e_s = _scale_tc(e_raw, b_raw)
  return stage_fn(e_s, hed, src, zeros)


def kernel(x, edge_index, params):
  src = edge_index[0]
  hed = edge_index[1]

  cbn = 1.0 / jnp.sqrt(jnp.float32(1.0 + EPSV))
  ws, bs = [], []
  for i in range(1, 8):
    g = params['bn%d_g' % i] * cbn
    wc = params['conv%d_W' % i] * g[None, :]
    bc = params['conv%d_b' % i] * g + params['bn%d_b' % i]
    w = jnp.concatenate([wc, params['sk%d_W' % i]], axis=1)
    b = jnp.concatenate([bc, params['sk%d_b' % i]])
    fin = w.shape[0]
    nct = 4 if fin == 96 else 2
    ws.append(w.reshape(nct, fin // nct, w.shape[1]))
    bs.append(b.reshape(1, -1))

  wm = params['conv_mix_W']
  w8 = wm[:80].reshape(2, 40, 48)
  w1 = wm[80:].reshape(4, 24, 48)
  bm = params['conv_mix_b'].reshape(1, -1)

  x_chunks = _split_x(x)

  z24 = jnp.zeros((EXP_ROWS, 24), jnp.float32)
  z40 = jnp.zeros((EXP_ROWS, 40), jnp.float32)
  z8 = jnp.zeros((EXP_ROWS, DEG_W), jnp.float32)
  ones8 = jnp.ones((DEG_K, DEG_W), jnp.float32)

  deg = _degrees(src, hed, ones8, z8)
  d_raw = deg[0]
  b_raw = deg[1]

  y1_raw = _sparse_pass(_stage_96, x_chunks, src, hed, z24, b_raw)
  h = _layer_tc(y1_raw, d_raw, ws[0], bs[0], 80)

  for i in range(1, 7):
    y = _sparse_pass(_stage_80, h, src, hed, z40, b_raw)
    h = _layer_tc(y, d_raw, ws[i], bs[i], 80)
  y8_raw = _sparse_pass(_stage_80, h, src, hed, z40, b_raw)
  return _final_tc(y8_raw, y1_raw, d_raw, w8, w1, bm)
